# Initial kernel scaffold; baseline (speedup 1.0000x reference)
import functools

import jax
import jax.numpy as jnp
from jax.experimental import pallas as pl
from jax.experimental.pallas import tpu as pltpu

N = 10000
E = 160000
D = 128
A = 4

_BN = 1000
_BE = 2048


def _silu(v):
    return v * jax.nn.sigmoid(v)


def _zmul(attr, x):
    return jnp.concatenate([attr[:, j:j + 1] * x for j in range(A)], axis=1)


def _wc(W):
    return W.transpose(2, 1, 0).reshape(W.shape[2] * W.shape[1], W.shape[0])


def _emb_body(x_ref, na_ref, w_ref, b_ref, o_ref):
    z = _zmul(na_ref[...], x_ref[...])
    o_ref[...] = jnp.dot(z, w_ref[...], preferred_element_type=jnp.float32) + b_ref[...]


def _edge_body(xi_ref, xj_ref, ea_ref, w1_ref, b1_ref, w2_ref, b2_ref, o_ref):
    ea = ea_ref[...]
    xcat = jnp.concatenate([xi_ref[...], xj_ref[...]], axis=1)
    m = _silu(jnp.dot(_zmul(ea, xcat), w1_ref[...],
                      preferred_element_type=jnp.float32) + b1_ref[...])
    o_ref[...] = _silu(jnp.dot(_zmul(ea, m), w2_ref[...],
                               preferred_element_type=jnp.float32) + b2_ref[...])


def _update_body(h_ref, agg_ref, na_ref, w1_ref, b1_ref, w2_ref, b2_ref, o_ref):
    na = na_ref[...]
    h = h_ref[...]
    hcat = jnp.concatenate([h, agg_ref[...]], axis=1)
    u = _silu(jnp.dot(_zmul(na, hcat), w1_ref[...],
                      preferred_element_type=jnp.float32) + b1_ref[...])
    o_ref[...] = h + jnp.dot(_zmul(na, u), w2_ref[...],
                             preferred_element_type=jnp.float32) + b2_ref[...]


def _head_body(h_ref, na_ref, w1_ref, b1_ref, w2_ref, b2_ref, o_ref):
    na = na_ref[...]
    u = _silu(jnp.dot(_zmul(na, h_ref[...]), w1_ref[...],
                      preferred_element_type=jnp.float32) + b1_ref[...])
    o_ref[...] = jnp.dot(_zmul(na, u), w2_ref[...],
                         preferred_element_type=jnp.float32) + b2_ref[...]


def _row_spec(bn, f):
    return pl.BlockSpec((bn, f), lambda i: (i, 0))


def _const_spec(shape):
    return pl.BlockSpec(shape, lambda i: tuple(0 for _ in shape))


def _call_rows(body, n, bn, ins, row_widths, consts, out_width):
    grid = n // bn
    in_specs = ([_row_spec(bn, w) for w in row_widths]
                + [_const_spec(c.shape) for c in consts])
    return pl.pallas_call(
        body,
        grid=(grid,),
        in_specs=in_specs,
        out_specs=_row_spec(bn, out_width),
        out_shape=jax.ShapeDtypeStruct((n, out_width), jnp.float32),
    )(*ins, *consts)


def kernel(x, edge_index, edge_attr, node_attr, batch, W_emb, b_emb,
           Wm1, bm1, Wm2, bm2, Wu1, bu1, Wu2, bu2, Wp1, bp1, Wp2, bp2):
    L = Wm1.shape[0]
    na = node_attr.at[:, 0].set(1.0)
    src = edge_index[0]
    dst = edge_index[1]

    w_emb = _wc(W_emb)
    b_embr = b_emb.reshape(1, D)

    h = _call_rows(_emb_body, N, _BN, [x, na], [D, A], [w_emb, b_embr], D)

    for l in range(L):
        w1 = _wc(Wm1[l])
        b1 = bm1[l].reshape(1, D)
        w2 = _wc(Wm2[l])
        b2 = bm2[l].reshape(1, D)
        wu1 = _wc(Wu1[l])
        bu1r = bu1[l].reshape(1, D)
        wu2 = _wc(Wu2[l])
        bu2r = bu2[l].reshape(1, D)

        xi = h[dst]
        xj = h[src]
        m2 = _call_rows(_edge_body, E, _BE, [xi, xj, edge_attr], [D, D, A],
                        [w1, b1, w2, b2], D)
        agg = jax.ops.segment_sum(m2, dst, num_segments=N)
        h = _call_rows(_update_body, N, _BN, [h, agg, na], [D, D, A],
                       [wu1, bu1r, wu2, bu2r], D)

    wp1 = _wc(Wp1)
    wp2 = _wc(Wp2)
    h = _call_rows(_head_body, N, _BN, [h, na], [D, A],
                   [wp1, bp1.reshape(1, D), wp2, bp2.reshape(1, D)], D)
    return h


# speedup vs baseline: 1.0922x; 1.0922x over previous
import functools

import jax
import jax.numpy as jnp
from jax.experimental import pallas as pl
from jax.experimental.pallas import tpu as pltpu

N = 10000
E = 160000
D = 128
A = 4

_BN = 1000
_BE = 2000


def _silu(v):
    return v * jax.nn.sigmoid(v)


def _zmul(attr, x):
    return jnp.concatenate([attr[:, j:j + 1] * x for j in range(A)], axis=1)


def _wc(W):
    return W.transpose(2, 1, 0).reshape(W.shape[2] * W.shape[1], W.shape[0])


def _emb_body(x_ref, na_ref, w_ref, b_ref, o_ref):
    z = _zmul(na_ref[...], x_ref[...])
    o_ref[...] = jnp.dot(z, w_ref[...], preferred_element_type=jnp.float32) + b_ref[...]


def _edge_body(xi_ref, xj_ref, ea_ref, w1_ref, b1_ref, w2_ref, b2_ref, o_ref):
    ea = ea_ref[...]
    xcat = jnp.concatenate([xi_ref[...], xj_ref[...]], axis=1)
    m = _silu(jnp.dot(_zmul(ea, xcat), w1_ref[...],
                      preferred_element_type=jnp.float32) + b1_ref[...])
    o_ref[...] = _silu(jnp.dot(_zmul(ea, m), w2_ref[...],
                               preferred_element_type=jnp.float32) + b2_ref[...])


def _update_body(h_ref, agg_ref, na_ref, w1_ref, b1_ref, w2_ref, b2_ref, o_ref):
    na = na_ref[...]
    h = h_ref[...]
    hcat = jnp.concatenate([h, agg_ref[...]], axis=1)
    u = _silu(jnp.dot(_zmul(na, hcat), w1_ref[...],
                      preferred_element_type=jnp.float32) + b1_ref[...])
    o_ref[...] = h + jnp.dot(_zmul(na, u), w2_ref[...],
                             preferred_element_type=jnp.float32) + b2_ref[...]


def _head_body(h_ref, na_ref, w1_ref, b1_ref, w2_ref, b2_ref, o_ref):
    na = na_ref[...]
    u = _silu(jnp.dot(_zmul(na, h_ref[...]), w1_ref[...],
                      preferred_element_type=jnp.float32) + b1_ref[...])
    o_ref[...] = jnp.dot(_zmul(na, u), w2_ref[...],
                         preferred_element_type=jnp.float32) + b2_ref[...]


def _row_spec(bn, f):
    return pl.BlockSpec((bn, f), lambda i: (i, 0))


def _const_spec(shape):
    return pl.BlockSpec(shape, lambda i: tuple(0 for _ in shape))


def _call_rows(body, n, bn, ins, row_widths, consts, out_width):
    grid = n // bn
    in_specs = ([_row_spec(bn, w) for w in row_widths]
                + [_const_spec(c.shape) for c in consts])
    return pl.pallas_call(
        body,
        grid=(grid,),
        in_specs=in_specs,
        out_specs=_row_spec(bn, out_width),
        out_shape=jax.ShapeDtypeStruct((n, out_width), jnp.float32),
    )(*ins, *consts)


def kernel(x, edge_index, edge_attr, node_attr, batch, W_emb, b_emb,
           Wm1, bm1, Wm2, bm2, Wu1, bu1, Wu2, bu2, Wp1, bp1, Wp2, bp2):
    L = Wm1.shape[0]
    na = node_attr.at[:, 0].set(1.0)
    src = edge_index[0]
    dst = edge_index[1]

    w_emb = _wc(W_emb)
    b_embr = b_emb.reshape(1, D)

    h = _call_rows(_emb_body, N, _BN, [x, na], [D, A], [w_emb, b_embr], D)

    for l in range(L):
        w1 = _wc(Wm1[l])
        b1 = bm1[l].reshape(1, D)
        w2 = _wc(Wm2[l])
        b2 = bm2[l].reshape(1, D)
        wu1 = _wc(Wu1[l])
        bu1r = bu1[l].reshape(1, D)
        wu2 = _wc(Wu2[l])
        bu2r = bu2[l].reshape(1, D)

        xi = h[dst]
        xj = h[src]
        m2 = _call_rows(_edge_body, E, _BE, [xi, xj, edge_attr], [D, D, A],
                        [w1, b1, w2, b2], D)
        agg = jax.ops.segment_sum(m2, dst, num_segments=N)
        h = _call_rows(_update_body, N, _BN, [h, agg, na], [D, D, A],
                       [wu1, bu1r, wu2, bu2r], D)

    wp1 = _wc(Wp1)
    wp2 = _wc(Wp2)
    h = _call_rows(_head_body, N, _BN, [h, na], [D, A],
                   [wp1, bp1.reshape(1, D), wp2, bp2.reshape(1, D)], D)
    return h
